# baseline (speedup 1.0000x reference)
import jax
import jax.numpy as jnp
from jax import lax
from jax.experimental import pallas as pl
from jax.experimental.pallas import tpu as pltpu
from jax.experimental.pallas import tpu_sc as plsc

N = 10000
E = 320000
D = 128
NC = 2
NS = 16
NW = NC * NS
E2 = 327680
EPW = E2 // NW
ERW = E // NW
K = 128
NCHUNK = EPW // K
WCH = 8
NWIN = NCHUNK // WCH
NB = 2
NQ = NCHUNK // NB
QPW = WCH // NB
NP = N + 16
RPT = 624
TAIL = NP - NS * RPT
TOFF = NS * RPT


def _make_agg(with_deg):
    mesh = plsc.VectorSubcoreMesh(core_axis_name="c", subcore_axis_name="s",
                                  num_cores=NC, num_subcores=NS)
    out_type = [jax.ShapeDtypeStruct((NC * NP, D), jnp.float32)]
    if with_deg:
        out_type.append(jax.ShapeDtypeStruct((NW, NP), jnp.float32))
    scratch = [
        pltpu.VMEM((2, WCH, K), jnp.int32),
        pltpu.VMEM((2, WCH, K), jnp.int32),
        pltpu.VMEM_SHARED((NP, D), jnp.float32),
        pltpu.SemaphoreType.DMA,
    ]
    scratch += [pltpu.VMEM((K, D), jnp.float32) for _ in range(NB)]
    scratch += [pltpu.SemaphoreType.DMA for _ in range(NB)]
    scratch += [pltpu.SemaphoreType.DMA for _ in range(NB)]
    if with_deg:
        scratch += [pltpu.VMEM((NP,), jnp.float32)]

    def body(*refs):
        if with_deg:
            (table, srcq, dstq, z2d, z1d, acc_out, deg_out,
             src_w, dst_w, acc_sh, sem_i, *rest) = refs
            deg_v = rest[-1]
            rest = rest[:-1]
        else:
            (table, srcq, dstq, z2d, acc_out,
             src_w, dst_w, acc_sh, sem_i, *rest) = refs
        rows = rest[:NB]
        sem_g = rest[NB:2 * NB]
        sem_s = rest[2 * NB:3 * NB]
        c = lax.axis_index("c")
        s = lax.axis_index("s")
        wid = c * NS + s

        pltpu.sync_copy(z2d, acc_sh.at[pl.ds(s * RPT, RPT)])

        @pl.when(s == 0)
        def _zero_tail():
            pltpu.sync_copy(z2d.at[pl.ds(0, TAIL)], acc_sh.at[pl.ds(TOFF, TAIL)])

        if with_deg:
            pltpu.sync_copy(z1d, deg_v)
        ones = jnp.full((16,), 1.0, jnp.float32)

        def refill_start(w, par):
            pltpu.async_copy(srcq.at[wid, w], src_w.at[par], sem_i)
            pltpu.async_copy(dstq.at[wid, w], dst_w.at[par], sem_i)

        def refill_wait(w, par):
            pltpu.make_async_copy(srcq.at[wid, w], src_w.at[par], sem_i).wait()
            pltpu.make_async_copy(dstq.at[wid, w], dst_w.at[par], sem_i).wait()

        def gath(par, jloc, b):
            return pltpu.make_async_copy(table.at[src_w.at[par, jloc]],
                                         rows[b], sem_g[b])

        def scat_start(par, jloc, b):
            pltpu.async_copy(rows[b], acc_sh.at[dst_w.at[par, jloc]],
                             sem_s[b], add=True)

        def scat_wait(par, jloc, b):
            pltpu.make_async_copy(rows[b], acc_sh.at[dst_w.at[par, jloc]],
                                  sem_s[b]).wait()

        def count_deg(par, jloc):
            if with_deg:
                for t in range(K // 16):
                    idx = dst_w[par, jloc, pl.ds(t * 16, 16)]
                    plsc.addupdate_scatter(deg_v, [idx], ones)

        refill_start(0, 0)
        refill_wait(0, 0)
        for b in range(NB):
            gath(0, b, b).start()
        plsc.subcore_barrier()

        def quad(q, carry):
            w = q // QPW
            par = lax.rem(w, 2)
            qloc = lax.rem(q, QPW)
            j0 = qloc * NB

            @pl.when(qloc == 0)
            def _():
                @pl.when(w + 1 < NWIN)
                def _():
                    refill_start(w + 1, 1 - par)

            for b in range(NB):
                gath(par, j0 + b, b).wait()
                count_deg(par, j0 + b)
                scat_start(par, j0 + b, b)

            @pl.when(qloc == QPW - 1)
            def _():
                @pl.when(w + 1 < NWIN)
                def _():
                    refill_wait(w + 1, 1 - par)

            @pl.when(q + 1 < NQ)
            def _():
                npar = lax.rem((q + 1) // QPW, 2)
                nj0 = lax.rem(q + 1, QPW) * NB
                for b in range(NB):
                    scat_wait(par, j0 + b, b)
                    gath(npar, nj0 + b, b).start()
            return carry
        lax.fori_loop(0, NQ, quad, 0)
        lpar = ((NQ - 1) // QPW) % 2
        lj0 = ((NQ - 1) % QPW) * NB
        for b in range(NB):
            scat_wait(lpar, lj0 + b, b)

        if with_deg:
            pltpu.sync_copy(deg_v, deg_out.at[wid])

        plsc.subcore_barrier()
        pltpu.sync_copy(acc_sh.at[pl.ds(s * RPT, RPT)],
                        acc_out.at[pl.ds(c * NP + s * RPT, RPT)])

        @pl.when(s == 0)
        def _write_tail():
            pltpu.sync_copy(acc_sh.at[pl.ds(TOFF, TAIL)],
                            acc_out.at[pl.ds(c * NP + TOFF, TAIL)])

    return pl.kernel(body, out_type=tuple(out_type), mesh=mesh,
                     scratch_types=tuple(scratch),
                     compiler_params=pltpu.CompilerParams(needs_layout_passes=False))


_agg_deg = _make_agg(True)
_agg = _make_agg(False)

BLK = 1000


def _tc1_body(x_ref, acc_ref, degp_ref, ws_ref, wn_ref, b_ref, h_ref, dinv_ref):
    deg = jnp.sum(degp_ref[...], axis=1)
    dinv = 1.0 / jnp.maximum(deg, 1.0)
    hn = (acc_ref[0] + acc_ref[1]) * dinv[:, None]
    h = (jnp.dot(x_ref[...], ws_ref[...], preferred_element_type=jnp.float32)
         + jnp.dot(hn, wn_ref[...], preferred_element_type=jnp.float32)
         + b_ref[...])
    h_ref[...] = jnp.maximum(h, 0.0)
    dinv_ref[...] = dinv[:, None]


def _tc2_body(h_ref, acc_ref, dinv_ref, ws_ref, wn_ref, b_ref, out_ref):
    hn = (acc_ref[0] + acc_ref[1]) * dinv_ref[...]
    out_ref[...] = (jnp.dot(h_ref[...], ws_ref[...], preferred_element_type=jnp.float32)
                    + jnp.dot(hn, wn_ref[...], preferred_element_type=jnp.float32)
                    + b_ref[...])


def _tc1(x, acc, degp_t, ws, wn, b):
    grid = (N // BLK,)
    return pl.pallas_call(
        _tc1_body,
        grid=grid,
        in_specs=[
            pl.BlockSpec((BLK, D), lambda i: (i, 0)),
            pl.BlockSpec((NC, BLK, D), lambda i: (0, i, 0)),
            pl.BlockSpec((BLK, NW), lambda i: (i, 0)),
            pl.BlockSpec((D, D), lambda i: (0, 0)),
            pl.BlockSpec((D, D), lambda i: (0, 0)),
            pl.BlockSpec((1, D), lambda i: (0, 0)),
        ],
        out_specs=[
            pl.BlockSpec((BLK, D), lambda i: (i, 0)),
            pl.BlockSpec((BLK, 1), lambda i: (i, 0)),
        ],
        out_shape=[
            jax.ShapeDtypeStruct((N, D), jnp.float32),
            jax.ShapeDtypeStruct((N, 1), jnp.float32),
        ],
    )(x, acc, degp_t, ws, wn, b)


def _tc2(h, acc, dinv, ws, wn, b):
    grid = (N // BLK,)
    return pl.pallas_call(
        _tc2_body,
        grid=grid,
        in_specs=[
            pl.BlockSpec((BLK, D), lambda i: (i, 0)),
            pl.BlockSpec((NC, BLK, D), lambda i: (0, i, 0)),
            pl.BlockSpec((BLK, 1), lambda i: (i, 0)),
            pl.BlockSpec((D, D), lambda i: (0, 0)),
            pl.BlockSpec((D, D), lambda i: (0, 0)),
            pl.BlockSpec((1, D), lambda i: (0, 0)),
        ],
        out_specs=pl.BlockSpec((BLK, D), lambda i: (i, 0)),
        out_shape=jax.ShapeDtypeStruct((N, D), jnp.float32),
    )(h, acc, dinv, ws, wn, b)


def kernel(x, edge_index, W_self1, W_neigh1, b1, W_self2, W_neigh2, b2):
    src = edge_index[0].astype(jnp.int32)
    dst = edge_index[1].astype(jnp.int32)
    ppw = EPW - ERW
    pad_src = jnp.zeros((NW, ppw), jnp.int32)
    pad_dst = jnp.broadcast_to(N + (jnp.arange(ppw, dtype=jnp.int32) % 16),
                               (NW, ppw))
    srcq = jnp.concatenate([src.reshape(NW, ERW), pad_src],
                           axis=1).reshape(NW, NWIN, WCH, K)
    dstq = jnp.concatenate([dst.reshape(NW, ERW), pad_dst],
                           axis=1).reshape(NW, NWIN, WCH, K)
    z2d = jnp.zeros((RPT, D), jnp.float32)
    z1d = jnp.zeros((NP,), jnp.float32)

    acc1, degp = _agg_deg(x, srcq, dstq, z2d, z1d)
    h1, dinv = _tc1(x, acc1.reshape(NC, NP, D), degp.T,
                    W_self1.T, W_neigh1.T, b1.reshape(1, D))
    acc2, = _agg(h1, srcq, dstq, z2d)
    out = _tc2(h1, acc2.reshape(NC, NP, D), dinv,
               W_self2.T, W_neigh2.T, b2.reshape(1, D))
    return out


# speedup vs baseline: 1.2619x; 1.2619x over previous
import jax
import jax.numpy as jnp
from jax import lax
from jax.experimental import pallas as pl
from jax.experimental.pallas import tpu as pltpu
from jax.experimental.pallas import tpu_sc as plsc

N = 10000
E = 320000
D = 128
NC = 2
NS = 16
NW = NC * NS
E2 = 327680
EPW = E2 // NW
ERW = E // NW
K = 128
NCHUNK = EPW // K
WCH = 8
NWIN = NCHUNK // WCH
NB = 2
QPW = WCH // NB
NP = N + 16
RPT = 624
TAIL = NP - NS * RPT
TOFF = NS * RPT


def _make_agg(with_deg):
    mesh = plsc.VectorSubcoreMesh(core_axis_name="c", subcore_axis_name="s",
                                  num_cores=NC, num_subcores=NS)
    out_type = [jax.ShapeDtypeStruct((NC * NP, D), jnp.float32)]
    if with_deg:
        out_type.append(jax.ShapeDtypeStruct((NW, NP), jnp.float32))
    scratch = [
        pltpu.VMEM((2, WCH, K), jnp.int32),
        pltpu.VMEM((2, WCH, K), jnp.int32),
        pltpu.VMEM_SHARED((NP, D), jnp.float32),
        pltpu.SemaphoreType.DMA,
    ]
    scratch += [pltpu.VMEM((K, D), jnp.float32) for _ in range(NB)]
    scratch += [pltpu.SemaphoreType.DMA for _ in range(NB)]
    if with_deg:
        scratch += [pltpu.VMEM((NP,), jnp.float32)]

    def body(*refs):
        if with_deg:
            (table, srcq, dstq, z2d, z1d, acc_out, deg_out,
             src_w, dst_w, acc_sh, sem_i, *rest) = refs
            deg_v = rest[-1]
            rest = rest[:-1]
        else:
            (table, srcq, dstq, z2d, acc_out,
             src_w, dst_w, acc_sh, sem_i, *rest) = refs
        rows = rest[:NB]
        sem_g = rest[NB:2 * NB]
        c = lax.axis_index("c")
        s = lax.axis_index("s")
        wid = c * NS + s

        pltpu.sync_copy(z2d, acc_sh.at[pl.ds(s * RPT, RPT)])

        @pl.when(s == 0)
        def _zero_tail():
            pltpu.sync_copy(z2d.at[pl.ds(0, TAIL)], acc_sh.at[pl.ds(TOFF, TAIL)])

        if with_deg:
            pltpu.sync_copy(z1d, deg_v)
        ones = jnp.full((16,), 1.0, jnp.float32)

        def refill_start(w, par):
            pltpu.async_copy(srcq.at[wid, w], src_w.at[par], sem_i)
            pltpu.async_copy(dstq.at[wid, w], dst_w.at[par], sem_i)

        def refill_wait(w, par):
            pltpu.make_async_copy(srcq.at[wid, w], src_w.at[par], sem_i).wait()
            pltpu.make_async_copy(dstq.at[wid, w], dst_w.at[par], sem_i).wait()

        def gath(par, jloc, b):
            return pltpu.make_async_copy(table.at[src_w.at[par, jloc]],
                                         rows[b], sem_g[b])

        def count_deg(par, jloc):
            if with_deg:
                for t in range(K // 16):
                    idx = dst_w[par, jloc, pl.ds(t * 16, 16)]
                    plsc.addupdate_scatter(deg_v, [idx], ones)

        def consume(par, j0, b):
            gath(par, j0 + b, b).wait()
            count_deg(par, j0 + b)
            pltpu.sync_copy(rows[b], acc_sh.at[dst_w.at[par, j0 + b]],
                            add=True)

        refill_start(0, 0)
        refill_wait(0, 0)
        for b in range(NB):
            gath(0, b, b).start()
        plsc.subcore_barrier()

        for w in range(NWIN):
            par = w % 2
            if w + 1 < NWIN:
                refill_start(w + 1, 1 - par)

            def quad(qloc, carry, par=par):
                j0 = qloc * NB
                for b in range(NB):
                    consume(par, j0, b)
                    gath(par, j0 + NB + b, b).start()
                return carry
            lax.fori_loop(0, QPW - 1, quad, 0)

            if w + 1 < NWIN:
                refill_wait(w + 1, 1 - par)
            for b in range(NB):
                consume(par, WCH - NB, b)
                if w + 1 < NWIN:
                    gath(1 - par, b, b).start()

        if with_deg:
            pltpu.sync_copy(deg_v, deg_out.at[wid])

        plsc.subcore_barrier()
        pltpu.sync_copy(acc_sh.at[pl.ds(s * RPT, RPT)],
                        acc_out.at[pl.ds(c * NP + s * RPT, RPT)])

        @pl.when(s == 0)
        def _write_tail():
            pltpu.sync_copy(acc_sh.at[pl.ds(TOFF, TAIL)],
                            acc_out.at[pl.ds(c * NP + TOFF, TAIL)])

    return pl.kernel(body, out_type=tuple(out_type), mesh=mesh,
                     scratch_types=tuple(scratch),
                     compiler_params=pltpu.CompilerParams(needs_layout_passes=False))


_agg_deg = _make_agg(True)
_agg = _make_agg(False)

BLK = 1000


def _tc1_body(x_ref, acc_ref, degp_ref, ws_ref, wn_ref, b_ref, h_ref, dinv_ref):
    deg = jnp.sum(degp_ref[...], axis=1)
    dinv = 1.0 / jnp.maximum(deg, 1.0)
    hn = (acc_ref[0] + acc_ref[1]) * dinv[:, None]
    h = (jnp.dot(x_ref[...], ws_ref[...], preferred_element_type=jnp.float32)
         + jnp.dot(hn, wn_ref[...], preferred_element_type=jnp.float32)
         + b_ref[...])
    h_ref[...] = jnp.maximum(h, 0.0)
    dinv_ref[...] = dinv[:, None]


def _tc2_body(h_ref, acc_ref, dinv_ref, ws_ref, wn_ref, b_ref, out_ref):
    hn = (acc_ref[0] + acc_ref[1]) * dinv_ref[...]
    out_ref[...] = (jnp.dot(h_ref[...], ws_ref[...], preferred_element_type=jnp.float32)
                    + jnp.dot(hn, wn_ref[...], preferred_element_type=jnp.float32)
                    + b_ref[...])


def _tc1(x, acc, degp_t, ws, wn, b):
    grid = (N // BLK,)
    return pl.pallas_call(
        _tc1_body,
        grid=grid,
        in_specs=[
            pl.BlockSpec((BLK, D), lambda i: (i, 0)),
            pl.BlockSpec((NC, BLK, D), lambda i: (0, i, 0)),
            pl.BlockSpec((BLK, NW), lambda i: (i, 0)),
            pl.BlockSpec((D, D), lambda i: (0, 0)),
            pl.BlockSpec((D, D), lambda i: (0, 0)),
            pl.BlockSpec((1, D), lambda i: (0, 0)),
        ],
        out_specs=[
            pl.BlockSpec((BLK, D), lambda i: (i, 0)),
            pl.BlockSpec((BLK, 1), lambda i: (i, 0)),
        ],
        out_shape=[
            jax.ShapeDtypeStruct((N, D), jnp.float32),
            jax.ShapeDtypeStruct((N, 1), jnp.float32),
        ],
    )(x, acc, degp_t, ws, wn, b)


def _tc2(h, acc, dinv, ws, wn, b):
    grid = (N // BLK,)
    return pl.pallas_call(
        _tc2_body,
        grid=grid,
        in_specs=[
            pl.BlockSpec((BLK, D), lambda i: (i, 0)),
            pl.BlockSpec((NC, BLK, D), lambda i: (0, i, 0)),
            pl.BlockSpec((BLK, 1), lambda i: (i, 0)),
            pl.BlockSpec((D, D), lambda i: (0, 0)),
            pl.BlockSpec((D, D), lambda i: (0, 0)),
            pl.BlockSpec((1, D), lambda i: (0, 0)),
        ],
        out_specs=pl.BlockSpec((BLK, D), lambda i: (i, 0)),
        out_shape=jax.ShapeDtypeStruct((N, D), jnp.float32),
    )(h, acc, dinv, ws, wn, b)


def kernel(x, edge_index, W_self1, W_neigh1, b1, W_self2, W_neigh2, b2):
    src = edge_index[0].astype(jnp.int32)
    dst = edge_index[1].astype(jnp.int32)
    ppw = EPW - ERW
    pad_src = jnp.zeros((NW, ppw), jnp.int32)
    pad_dst = jnp.broadcast_to(N + (jnp.arange(ppw, dtype=jnp.int32) % 16),
                               (NW, ppw))
    srcq = jnp.concatenate([src.reshape(NW, ERW), pad_src],
                           axis=1).reshape(NW, NWIN, WCH, K)
    dstq = jnp.concatenate([dst.reshape(NW, ERW), pad_dst],
                           axis=1).reshape(NW, NWIN, WCH, K)
    z2d = jnp.zeros((RPT, D), jnp.float32)
    z1d = jnp.zeros((NP,), jnp.float32)

    acc1, degp = _agg_deg(x, srcq, dstq, z2d, z1d)
    h1, dinv = _tc1(x, acc1.reshape(NC, NP, D), degp.T,
                    W_self1.T, W_neigh1.T, b1.reshape(1, D))
    acc2, = _agg(h1, srcq, dstq, z2d)
    out = _tc2(h1, acc2.reshape(NC, NP, D), dinv,
               W_self2.T, W_neigh2.T, b2.reshape(1, D))
    return out
